# baseline (speedup 1.0000x reference)
import jax
import jax.numpy as jnp
from jax import lax
from jax.experimental import pallas as pl
from jax.experimental.pallas import tpu as pltpu

POOL_SIZE = 64
LENGTH = 16
EMBED_DIM = 1024
TOP_K = 8
BATCH = 128
TAU = 5.0
NEG_INF = -3.0e38

NBP = 8
NKY = 4
BPC = LENGTH * EMBED_DIM // NBP
KYC = TOP_K * EMBED_DIM // NKY


def _body(cls_ref, pk_ref, prompt_ref, bp_hbm, sim_ref, keys_hbm, idx_ref,
          pool_ref, bp_v, keys_v, sems):
    cls = cls_ref[...]
    pk = pk_ref[...]
    eps = 1e-12
    xn = cls * lax.rsqrt(jnp.maximum(jnp.sum(cls * cls, axis=1, keepdims=True), eps))
    pn = pk * lax.rsqrt(jnp.maximum(jnp.sum(pk * pk, axis=1, keepdims=True), eps))
    sim = lax.dot_general(xn, pn, (((1,), (1,)), ((), ())),
                          preferred_element_type=jnp.float32)
    sim_ref[...] = sim
    z = (sim - jnp.max(sim, axis=1, keepdims=True)) * (1.0 / TAU)
    e = jnp.exp(z)
    w = e / jnp.sum(e, axis=1, keepdims=True)

    col = lax.broadcasted_iota(jnp.int32, (BATCH, POOL_SIZE), 1)
    kcol = lax.broadcasted_iota(jnp.int32, (BATCH, TOP_K), 1)
    vals = sim
    selected = jnp.zeros((BATCH, POOL_SIZE), dtype=jnp.bool_)
    idx_acc = jnp.zeros((BATCH, TOP_K), dtype=jnp.int32)
    for k in range(TOP_K):
        m = jnp.max(vals, axis=1, keepdims=True)
        cand = jnp.where(vals == m, col, POOL_SIZE)
        sel = jnp.min(cand, axis=1, keepdims=True)
        hit = col == sel
        vals = jnp.where(hit, NEG_INF, vals)
        selected = jnp.logical_or(selected, hit)
        idx_acc = jnp.where(kcol == k, sel, idx_acc)
        keys_v[:, k * EMBED_DIM:(k + 1) * EMBED_DIM] = jnp.dot(
            hit.astype(jnp.float32), pn, preferred_element_type=jnp.float32)
    idx_ref[...] = idx_acc
    pool_ref[...] = jnp.sum(selected.astype(jnp.float32), axis=0, keepdims=True)

    key_copies = []
    for c in range(NKY):
        cp = pltpu.make_async_copy(
            keys_v.at[:, c * KYC:(c + 1) * KYC],
            keys_hbm.at[:, c * KYC:(c + 1) * KYC], sems.at[NBP + c])
        cp.start()
        key_copies.append(cp)

    bp_copies = []
    for c in range(NBP):
        bp_v[:, c * BPC:(c + 1) * BPC] = jnp.dot(
            w, prompt_ref[:, c * BPC:(c + 1) * BPC],
            preferred_element_type=jnp.float32)
        cp = pltpu.make_async_copy(
            bp_v.at[:, c * BPC:(c + 1) * BPC],
            bp_hbm.at[:, c * BPC:(c + 1) * BPC], sems.at[c])
        cp.start()
        bp_copies.append(cp)

    for cp in key_copies:
        cp.wait()
    for cp in bp_copies:
        cp.wait()


def kernel(x_embed, cls_features, prompt, prompt_key, cur_task, train_mode):
    del x_embed, cur_task, train_mode
    prompt_flat = prompt.reshape(POOL_SIZE, LENGTH * EMBED_DIM)
    bp, sim, keys, idx, pool = pl.pallas_call(
        _body,
        in_specs=[
            pl.BlockSpec(memory_space=pltpu.VMEM),
            pl.BlockSpec(memory_space=pltpu.VMEM),
            pl.BlockSpec(memory_space=pltpu.VMEM),
        ],
        out_specs=(
            pl.BlockSpec(memory_space=pl.ANY),
            pl.BlockSpec(memory_space=pltpu.VMEM),
            pl.BlockSpec(memory_space=pl.ANY),
            pl.BlockSpec(memory_space=pltpu.VMEM),
            pl.BlockSpec(memory_space=pltpu.VMEM),
        ),
        out_shape=(
            jax.ShapeDtypeStruct((BATCH, LENGTH * EMBED_DIM), jnp.float32),
            jax.ShapeDtypeStruct((BATCH, POOL_SIZE), jnp.float32),
            jax.ShapeDtypeStruct((BATCH, TOP_K * EMBED_DIM), jnp.float32),
            jax.ShapeDtypeStruct((BATCH, TOP_K), jnp.int32),
            jax.ShapeDtypeStruct((1, POOL_SIZE), jnp.float32),
        ),
        scratch_shapes=[
            pltpu.VMEM((BATCH, LENGTH * EMBED_DIM), jnp.float32),
            pltpu.VMEM((BATCH, TOP_K * EMBED_DIM), jnp.float32),
            pltpu.SemaphoreType.DMA((NBP + NKY,)),
        ],
    )(cls_features, prompt_key, prompt_flat)
    return (bp.reshape(BATCH, LENGTH, EMBED_DIM), sim,
            keys.reshape(BATCH, TOP_K, EMBED_DIM), idx, pool.reshape(POOL_SIZE))


# speedup vs baseline: 2.4926x; 2.4926x over previous
import jax
import jax.numpy as jnp
from jax import lax
from jax.experimental import pallas as pl

POOL_SIZE = 64
LENGTH = 16
EMBED_DIM = 1024
TOP_K = 8
BATCH = 128
TAU = 5.0
NEG_INF = -3.0e38


def _tc_body(cls_ref, pk_ref, prompt_ref, bp_ref, sim_ref, keys_ref, idx_ref,
             pool_ref):
    cls = cls_ref[...]
    pk = pk_ref[...]
    eps = 1e-12
    xn = cls * lax.rsqrt(jnp.maximum(jnp.sum(cls * cls, axis=1, keepdims=True), eps))
    pn = pk * lax.rsqrt(jnp.maximum(jnp.sum(pk * pk, axis=1, keepdims=True), eps))
    sim = lax.dot_general(xn, pn, (((1,), (1,)), ((), ())),
                          preferred_element_type=jnp.float32)
    sim_ref[...] = sim

    z = (sim - jnp.max(sim, axis=1, keepdims=True)) * (1.0 / TAU)
    e = jnp.exp(z)
    w = e / jnp.sum(e, axis=1, keepdims=True)

    col = lax.broadcasted_iota(jnp.int32, (BATCH, POOL_SIZE), 1)
    kcol = lax.broadcasted_iota(jnp.int32, (BATCH, TOP_K), 1)
    vals = sim
    selected = jnp.zeros((BATCH, POOL_SIZE), dtype=jnp.bool_)
    idx_acc = jnp.zeros((BATCH, TOP_K), dtype=jnp.int32)
    for k in range(TOP_K):
        m = jnp.max(vals, axis=1, keepdims=True)
        cand = jnp.where(vals == m, col, POOL_SIZE)
        sel = jnp.min(cand, axis=1, keepdims=True)
        hit = col == sel
        vals = jnp.where(hit, NEG_INF, vals)
        selected = jnp.logical_or(selected, hit)
        idx_acc = jnp.where(kcol == k, sel, idx_acc)
        keys_ref[:, k, :] = jnp.dot(hit.astype(jnp.float32), pn,
                                    preferred_element_type=jnp.float32)
    idx_ref[...] = idx_acc
    pool_ref[...] = jnp.sum(selected.astype(jnp.float32), axis=0,
                            keepdims=True)

    for l in range(LENGTH):
        bp_ref[:, l, :] = jnp.dot(w, prompt_ref[:, l, :],
                                  preferred_element_type=jnp.float32)


def kernel(x_embed, cls_features, prompt, prompt_key, cur_task, train_mode):
    del x_embed, cur_task, train_mode
    bp, sim, keys, idx, pool = pl.pallas_call(
        _tc_body,
        out_shape=(
            jax.ShapeDtypeStruct((BATCH, LENGTH, EMBED_DIM), jnp.float32),
            jax.ShapeDtypeStruct((BATCH, POOL_SIZE), jnp.float32),
            jax.ShapeDtypeStruct((BATCH, TOP_K, EMBED_DIM), jnp.float32),
            jax.ShapeDtypeStruct((BATCH, TOP_K), jnp.int32),
            jax.ShapeDtypeStruct((1, POOL_SIZE), jnp.float32),
        ),
    )(cls_features, prompt_key, prompt)
    return (bp, sim, keys, idx, pool.reshape(POOL_SIZE))


# speedup vs baseline: 2.7059x; 1.0856x over previous
import jax
import jax.numpy as jnp
from jax import lax
from jax.experimental import pallas as pl

POOL_SIZE = 64
LENGTH = 16
EMBED_DIM = 1024
TOP_K = 8
BATCH = 128
TAU = 5.0
NEG_INF = -3.0e38


def _tc_body(cls_ref, pk_ref, prompt_ref, bp_ref, sim_ref, keys_ref, idx_ref,
             pool_ref):
    cls = cls_ref[...]
    pk = pk_ref[...]
    eps = 1e-12
    xn = cls * lax.rsqrt(jnp.maximum(jnp.sum(cls * cls, axis=1, keepdims=True), eps))
    pn = pk * lax.rsqrt(jnp.maximum(jnp.sum(pk * pk, axis=1, keepdims=True), eps))
    sim = lax.dot_general(xn, pn, (((1,), (1,)), ((), ())),
                          preferred_element_type=jnp.float32)
    sim_ref[...] = sim

    z = (sim - jnp.max(sim, axis=1, keepdims=True)) * (1.0 / TAU)
    e = jnp.exp(z)
    w = e / jnp.sum(e, axis=1, keepdims=True)

    col = lax.broadcasted_iota(jnp.int32, (BATCH, POOL_SIZE), 1)
    kcol = lax.broadcasted_iota(jnp.int32, (BATCH, TOP_K), 1)
    vals = sim
    selected = jnp.zeros((BATCH, POOL_SIZE), dtype=jnp.bool_)
    idx_acc = jnp.zeros((BATCH, TOP_K), dtype=jnp.int32)
    for k in range(TOP_K):
        m = jnp.max(vals, axis=1, keepdims=True)
        cand = jnp.where(vals == m, col, POOL_SIZE)
        sel = jnp.min(cand, axis=1, keepdims=True)
        hit = col == sel
        vals = jnp.where(hit, NEG_INF, vals)
        selected = jnp.logical_or(selected, hit)
        idx_acc = jnp.where(kcol == k, sel, idx_acc)
        keys_ref[:, k, :] = jnp.dot(hit.astype(jnp.float32), pn,
                                    preferred_element_type=jnp.float32)
    idx_ref[...] = idx_acc
    pool_ref[...] = jnp.sum(selected.astype(jnp.float32), axis=0,
                            keepdims=True)

    bp_ref[...] = lax.dot_general(w, prompt_ref[...], (((1,), (0,)), ((), ())),
                                  preferred_element_type=jnp.float32)


def kernel(x_embed, cls_features, prompt, prompt_key, cur_task, train_mode):
    del x_embed, cur_task, train_mode
    bp, sim, keys, idx, pool = pl.pallas_call(
        _tc_body,
        out_shape=(
            jax.ShapeDtypeStruct((BATCH, LENGTH, EMBED_DIM), jnp.float32),
            jax.ShapeDtypeStruct((BATCH, POOL_SIZE), jnp.float32),
            jax.ShapeDtypeStruct((BATCH, TOP_K, EMBED_DIM), jnp.float32),
            jax.ShapeDtypeStruct((BATCH, TOP_K), jnp.int32),
            jax.ShapeDtypeStruct((1, POOL_SIZE), jnp.float32),
        ),
    )(cls_features, prompt_key, prompt)
    return (bp, sim, keys, idx, pool.reshape(POOL_SIZE))


# speedup vs baseline: 2.7361x; 1.0111x over previous
import jax
import jax.numpy as jnp
from jax import lax
from jax.experimental import pallas as pl

POOL_SIZE = 64
LENGTH = 16
EMBED_DIM = 1024
TOP_K = 8
BATCH = 128
TAU = 5.0
NEG_INF = -3.0e38


def _tc_body(cls_ref, pk_ref, prompt_ref, bp_ref, sim_ref, keys_ref, idx_ref,
             pool_ref):
    cls = cls_ref[...]
    pk = pk_ref[...]
    eps = 1e-12
    xn = cls * lax.rsqrt(jnp.maximum(jnp.sum(cls * cls, axis=1, keepdims=True), eps))
    pn = pk * lax.rsqrt(jnp.maximum(jnp.sum(pk * pk, axis=1, keepdims=True), eps))
    sim = lax.dot_general(xn, pn, (((1,), (1,)), ((), ())),
                          preferred_element_type=jnp.float32)
    sim_ref[...] = sim

    z = (sim - jnp.max(sim, axis=1, keepdims=True)) * (1.0 / TAU)
    e = jnp.exp(z)
    w = e / jnp.sum(e, axis=1, keepdims=True)

    col = lax.broadcasted_iota(jnp.int32, (BATCH, POOL_SIZE), 1)
    kcol = lax.broadcasted_iota(jnp.int32, (BATCH, TOP_K), 1)
    vals = sim
    selected = jnp.zeros((BATCH, POOL_SIZE), dtype=jnp.bool_)
    idx_acc = jnp.zeros((BATCH, TOP_K), dtype=jnp.int32)
    for k in range(TOP_K):
        m = jnp.max(vals, axis=1, keepdims=True)
        cand = jnp.where(vals == m, col, POOL_SIZE)
        sel = jnp.min(cand, axis=1, keepdims=True)
        hit = col == sel
        vals = jnp.where(hit, NEG_INF, vals)
        selected = jnp.logical_or(selected, hit)
        idx_acc = jnp.where(kcol == k, sel, idx_acc)
    idx_ref[...] = idx_acc
    oh = (idx_acc[:, :, None] ==
          lax.broadcasted_iota(jnp.int32, (BATCH, TOP_K, POOL_SIZE), 2)
          ).astype(jnp.float32)
    keys_ref[...] = lax.dot_general(oh, pn, (((2,), (0,)), ((), ())),
                                    preferred_element_type=jnp.float32)
    pool_ref[...] = jnp.sum(selected.astype(jnp.float32), axis=0,
                            keepdims=True)

    bp_ref[...] = lax.dot_general(w, prompt_ref[...], (((1,), (0,)), ((), ())),
                                  preferred_element_type=jnp.float32)


def kernel(x_embed, cls_features, prompt, prompt_key, cur_task, train_mode):
    del x_embed, cur_task, train_mode
    bp, sim, keys, idx, pool = pl.pallas_call(
        _tc_body,
        out_shape=(
            jax.ShapeDtypeStruct((BATCH, LENGTH, EMBED_DIM), jnp.float32),
            jax.ShapeDtypeStruct((BATCH, POOL_SIZE), jnp.float32),
            jax.ShapeDtypeStruct((BATCH, TOP_K, EMBED_DIM), jnp.float32),
            jax.ShapeDtypeStruct((BATCH, TOP_K), jnp.int32),
            jax.ShapeDtypeStruct((1, POOL_SIZE), jnp.float32),
        ),
    )(cls_features, prompt_key, prompt)
    return (bp, sim, keys, idx, pool.reshape(POOL_SIZE))


# speedup vs baseline: 2.7440x; 1.0029x over previous
import jax
import jax.numpy as jnp
from jax import lax
from jax.experimental import pallas as pl

POOL_SIZE = 64
LENGTH = 16
EMBED_DIM = 1024
TOP_K = 8
BATCH = 128
TAU = 5.0
NEG_INF = -3.0e38


def _tc_body(cls_ref, pk_ref, prompt_ref, bp_ref, sim_ref, keys_ref, idx_ref,
             pool_ref):
    cls = cls_ref[...]
    pk = pk_ref[...]
    eps = 1e-12
    xn = cls * lax.rsqrt(jnp.maximum(jnp.sum(cls * cls, axis=1, keepdims=True), eps))
    pn = pk * lax.rsqrt(jnp.maximum(jnp.sum(pk * pk, axis=1, keepdims=True), eps))
    sim = lax.dot_general(xn, pn, (((1,), (1,)), ((), ())),
                          preferred_element_type=jnp.float32)
    sim_ref[...] = sim

    z = (sim - jnp.max(sim, axis=1, keepdims=True)) * (1.0 / TAU)
    e = jnp.exp(z)
    w = e / jnp.sum(e, axis=1, keepdims=True)

    col = lax.broadcasted_iota(jnp.int32, (BATCH, POOL_SIZE), 1)
    kcol = lax.broadcasted_iota(jnp.int32, (BATCH, TOP_K), 1)
    vals = sim
    selected = jnp.zeros((BATCH, POOL_SIZE), dtype=jnp.bool_)
    idx_acc = jnp.zeros((BATCH, TOP_K), dtype=jnp.int32)
    for k in range(TOP_K):
        m = jnp.max(vals, axis=1, keepdims=True)
        cand = jnp.where(vals == m, col, POOL_SIZE)
        sel = jnp.min(cand, axis=1, keepdims=True)
        hit = col == sel
        vals = jnp.where(hit, NEG_INF, vals)
        selected = jnp.logical_or(selected, hit)
        idx_acc = jnp.where(kcol == k, sel, idx_acc)
    idx_ref[...] = idx_acc
    oh = (idx_acc[:, :, None] ==
          lax.broadcasted_iota(jnp.int32, (BATCH, TOP_K, POOL_SIZE), 2)
          ).astype(jnp.float32)
    keys_ref[...] = lax.dot_general(oh, pn, (((2,), (0,)), ((), ())),
                                    preferred_element_type=jnp.float32)
    pool_ref[...] = jnp.sum(selected.astype(jnp.float32), axis=0)

    bp_ref[...] = lax.dot_general(w, prompt_ref[...], (((1,), (0,)), ((), ())),
                                  preferred_element_type=jnp.float32)


def kernel(x_embed, cls_features, prompt, prompt_key, cur_task, train_mode):
    del x_embed, cur_task, train_mode
    bp, sim, keys, idx, pool = pl.pallas_call(
        _tc_body,
        out_shape=(
            jax.ShapeDtypeStruct((BATCH, LENGTH, EMBED_DIM), jnp.float32),
            jax.ShapeDtypeStruct((BATCH, POOL_SIZE), jnp.float32),
            jax.ShapeDtypeStruct((BATCH, TOP_K, EMBED_DIM), jnp.float32),
            jax.ShapeDtypeStruct((BATCH, TOP_K), jnp.int32),
            jax.ShapeDtypeStruct((POOL_SIZE,), jnp.float32),
        ),
    )(cls_features, prompt_key, prompt)
    return (bp, sim, keys, idx, pool)


# speedup vs baseline: 2.7485x; 1.0016x over previous
import jax
import jax.numpy as jnp
from jax import lax
from jax.experimental import pallas as pl
from jax.experimental.pallas import tpu as pltpu

POOL_SIZE = 64
LENGTH = 16
EMBED_DIM = 1024
TOP_K = 8
BATCH = 128
TAU = 5.0
NEG_INF = -3.0e38

NSTEP = 2
LCHUNK = LENGTH // NSTEP


def _tc_body(cls_ref, pk_ref, prompt_ref, bp_ref, sim_ref, keys_ref, idx_ref,
             pool_ref, w_ref, pn_ref, idxv_ref):
    j = pl.program_id(0)

    @pl.when(j == 0)
    def _route():
        cls = cls_ref[...]
        pk = pk_ref[...]
        eps = 1e-12
        xn = cls * lax.rsqrt(jnp.maximum(jnp.sum(cls * cls, axis=1, keepdims=True), eps))
        pn = pk * lax.rsqrt(jnp.maximum(jnp.sum(pk * pk, axis=1, keepdims=True), eps))
        pn_ref[...] = pn
        sim = lax.dot_general(xn, pn, (((1,), (1,)), ((), ())),
                              preferred_element_type=jnp.float32)
        sim_ref[...] = sim
        z = (sim - jnp.max(sim, axis=1, keepdims=True)) * (1.0 / TAU)
        e = jnp.exp(z)
        w_ref[...] = e / jnp.sum(e, axis=1, keepdims=True)
        col = lax.broadcasted_iota(jnp.int32, (BATCH, POOL_SIZE), 1)
        kcol = lax.broadcasted_iota(jnp.int32, (BATCH, TOP_K), 1)
        vals = sim
        selected = jnp.zeros((BATCH, POOL_SIZE), dtype=jnp.bool_)
        idx_acc = jnp.zeros((BATCH, TOP_K), dtype=jnp.int32)
        for k in range(TOP_K):
            m = jnp.max(vals, axis=1, keepdims=True)
            cand = jnp.where(vals == m, col, POOL_SIZE)
            sel = jnp.min(cand, axis=1, keepdims=True)
            hit = col == sel
            vals = jnp.where(hit, NEG_INF, vals)
            selected = jnp.logical_or(selected, hit)
            idx_acc = jnp.where(kcol == k, sel, idx_acc)
        idx_ref[...] = idx_acc
        idxv_ref[...] = idx_acc
        pool_ref[...] = jnp.sum(selected.astype(jnp.float32), axis=0)

    bp_ref[...] = lax.dot_general(w_ref[...], prompt_ref[...],
                                  (((1,), (0,)), ((), ())),
                                  preferred_element_type=jnp.float32)

    @pl.when(j == NSTEP - 1)
    def _keys():
        oh = (idxv_ref[...][:, :, None] ==
              lax.broadcasted_iota(jnp.int32, (BATCH, TOP_K, POOL_SIZE), 2)
              ).astype(jnp.float32)
        keys_ref[...] = lax.dot_general(oh, pn_ref[...],
                                        (((2,), (0,)), ((), ())),
                                        preferred_element_type=jnp.float32)


def kernel(x_embed, cls_features, prompt, prompt_key, cur_task, train_mode):
    del x_embed, cur_task, train_mode
    bp, sim, keys, idx, pool = pl.pallas_call(
        _tc_body,
        grid=(NSTEP,),
        in_specs=[
            pl.BlockSpec((BATCH, EMBED_DIM), lambda j: (0, 0)),
            pl.BlockSpec((POOL_SIZE, EMBED_DIM), lambda j: (0, 0)),
            pl.BlockSpec((POOL_SIZE, LCHUNK, EMBED_DIM), lambda j: (0, j, 0)),
        ],
        out_specs=(
            pl.BlockSpec((BATCH, LCHUNK, EMBED_DIM), lambda j: (0, j, 0)),
            pl.BlockSpec((BATCH, POOL_SIZE), lambda j: (0, 0)),
            pl.BlockSpec((BATCH, TOP_K, EMBED_DIM), lambda j: (0, 0, 0)),
            pl.BlockSpec((BATCH, TOP_K), lambda j: (0, 0)),
            pl.BlockSpec((POOL_SIZE,), lambda j: (0,)),
        ),
        out_shape=(
            jax.ShapeDtypeStruct((BATCH, LENGTH, EMBED_DIM), jnp.float32),
            jax.ShapeDtypeStruct((BATCH, POOL_SIZE), jnp.float32),
            jax.ShapeDtypeStruct((BATCH, TOP_K, EMBED_DIM), jnp.float32),
            jax.ShapeDtypeStruct((BATCH, TOP_K), jnp.int32),
            jax.ShapeDtypeStruct((POOL_SIZE,), jnp.float32),
        ),
        scratch_shapes=[
            pltpu.VMEM((BATCH, POOL_SIZE), jnp.float32),
            pltpu.VMEM((POOL_SIZE, EMBED_DIM), jnp.float32),
            pltpu.VMEM((BATCH, TOP_K), jnp.int32),
        ],
    )(cls_features, prompt_key, prompt)
    return (bp, sim, keys, idx, pool)


# speedup vs baseline: 3.3095x; 1.2041x over previous
import jax
import jax.numpy as jnp
from jax import lax
from jax.experimental import pallas as pl

POOL_SIZE = 64
LENGTH = 16
EMBED_DIM = 1024
TOP_K = 8
BATCH = 128
TAU = 5.0
NEG_INF = -3.0e38


def _tc_body(cls_ref, pk_ref, prompt_ref, bp_ref, sim_ref, keys_ref, idx_ref,
             pool_ref):
    cls = cls_ref[...]
    pk = pk_ref[...]
    eps = 1e-12
    xn = cls * lax.rsqrt(jnp.maximum(jnp.sum(cls * cls, axis=1, keepdims=True), eps))
    pn = pk * lax.rsqrt(jnp.maximum(jnp.sum(pk * pk, axis=1, keepdims=True), eps))
    sim = lax.dot_general(xn, pn, (((1,), (1,)), ((), ())),
                          preferred_element_type=jnp.float32)
    sim_ref[...] = lax.dot_general(pn, xn, (((1,), (1,)), ((), ())),
                                   preferred_element_type=jnp.float32)

    z = (sim - jnp.max(sim, axis=1, keepdims=True)) * (1.0 / TAU)
    e = jnp.exp(z)
    w = e / jnp.sum(e, axis=1, keepdims=True)

    col = lax.broadcasted_iota(jnp.int32, (BATCH, POOL_SIZE), 1)
    kcol = lax.broadcasted_iota(jnp.int32, (BATCH, TOP_K), 1)
    vals = sim
    selected = jnp.zeros((BATCH, POOL_SIZE), dtype=jnp.bool_)
    idx_acc = jnp.zeros((BATCH, TOP_K), dtype=jnp.int32)
    for k in range(TOP_K):
        m = jnp.max(vals, axis=1, keepdims=True)
        cand = jnp.where(vals == m, col, POOL_SIZE)
        sel = jnp.min(cand, axis=1, keepdims=True)
        hit = col == sel
        vals = jnp.where(hit, NEG_INF, vals)
        selected = jnp.logical_or(selected, hit)
        idx_acc = jnp.where(kcol == k, sel, idx_acc)
    idx_ref[...] = idx_acc.T
    oh = (idx_acc[:, :, None] ==
          lax.broadcasted_iota(jnp.int32, (BATCH, TOP_K, POOL_SIZE), 2)
          ).astype(jnp.float32)
    keys_ref[...] = lax.dot_general(oh, pn, (((2,), (0,)), ((), ())),
                                    preferred_element_type=jnp.float32)
    pool_ref[...] = jnp.sum(selected.astype(jnp.float32), axis=0)

    bp_ref[...] = lax.dot_general(w, prompt_ref[...], (((1,), (0,)), ((), ())),
                                  preferred_element_type=jnp.float32)


def kernel(x_embed, cls_features, prompt, prompt_key, cur_task, train_mode):
    del x_embed, cur_task, train_mode
    bp, sim, keys, idx, pool = pl.pallas_call(
        _tc_body,
        out_shape=(
            jax.ShapeDtypeStruct((BATCH, LENGTH, EMBED_DIM), jnp.float32),
            jax.ShapeDtypeStruct((POOL_SIZE, BATCH), jnp.float32),
            jax.ShapeDtypeStruct((BATCH, TOP_K, EMBED_DIM), jnp.float32),
            jax.ShapeDtypeStruct((TOP_K, BATCH), jnp.int32),
            jax.ShapeDtypeStruct((POOL_SIZE,), jnp.float32),
        ),
    )(cls_features, prompt_key, prompt)
    return (bp, sim.T, keys, idx.T, pool)


# speedup vs baseline: 3.3920x; 1.0249x over previous
import jax
import jax.numpy as jnp
from jax import lax
from jax.experimental import pallas as pl
from jax.experimental.pallas import tpu as pltpu

POOL_SIZE = 64
LENGTH = 16
EMBED_DIM = 1024
TOP_K = 8
BATCH = 128
TAU = 5.0
NEG_INF = -3.0e38

NSTEP = 2
LCHUNK = LENGTH // NSTEP


def _tc_body(cls_ref, pk_ref, prompt_ref, bp_ref, sim_ref, keys_ref, idx_ref,
             pool_ref, w_ref, pn_ref, idxv_ref):
    j = pl.program_id(0)

    @pl.when(j == 0)
    def _route():
        cls = cls_ref[...]
        pk = pk_ref[...]
        eps = 1e-12
        xn = cls * lax.rsqrt(jnp.maximum(jnp.sum(cls * cls, axis=1, keepdims=True), eps))
        pn = pk * lax.rsqrt(jnp.maximum(jnp.sum(pk * pk, axis=1, keepdims=True), eps))
        pn_ref[...] = pn
        sim = lax.dot_general(xn, pn, (((1,), (1,)), ((), ())),
                              preferred_element_type=jnp.float32)
        sim_ref[...] = lax.dot_general(pn, xn, (((1,), (1,)), ((), ())),
                                       preferred_element_type=jnp.float32)
        z = (sim - jnp.max(sim, axis=1, keepdims=True)) * (1.0 / TAU)
        e = jnp.exp(z)
        w_ref[...] = e / jnp.sum(e, axis=1, keepdims=True)
        col = lax.broadcasted_iota(jnp.int32, (BATCH, POOL_SIZE), 1)
        kcol = lax.broadcasted_iota(jnp.int32, (BATCH, TOP_K), 1)
        vals = sim
        selected = jnp.zeros((BATCH, POOL_SIZE), dtype=jnp.bool_)
        idx_acc = jnp.zeros((BATCH, TOP_K), dtype=jnp.int32)
        for k in range(TOP_K):
            m = jnp.max(vals, axis=1, keepdims=True)
            cand = jnp.where(vals == m, col, POOL_SIZE)
            sel = jnp.min(cand, axis=1, keepdims=True)
            hit = col == sel
            vals = jnp.where(hit, NEG_INF, vals)
            selected = jnp.logical_or(selected, hit)
            idx_acc = jnp.where(kcol == k, sel, idx_acc)
        idx_ref[...] = idx_acc.T
        idxv_ref[...] = idx_acc
        pool_ref[...] = jnp.sum(selected.astype(jnp.float32), axis=0)

    bp_ref[...] = lax.dot_general(w_ref[...], prompt_ref[...],
                                  (((1,), (0,)), ((), ())),
                                  preferred_element_type=jnp.float32)

    @pl.when(j == NSTEP - 1)
    def _keys():
        oh = (idxv_ref[...][:, :, None] ==
              lax.broadcasted_iota(jnp.int32, (BATCH, TOP_K, POOL_SIZE), 2)
              ).astype(jnp.float32)
        keys_ref[...] = lax.dot_general(oh, pn_ref[...],
                                        (((2,), (0,)), ((), ())),
                                        preferred_element_type=jnp.float32)


def kernel(x_embed, cls_features, prompt, prompt_key, cur_task, train_mode):
    del x_embed, cur_task, train_mode
    bp, sim, keys, idx, pool = pl.pallas_call(
        _tc_body,
        grid=(NSTEP,),
        in_specs=[
            pl.BlockSpec((BATCH, EMBED_DIM), lambda j: (0, 0)),
            pl.BlockSpec((POOL_SIZE, EMBED_DIM), lambda j: (0, 0)),
            pl.BlockSpec((POOL_SIZE, LCHUNK, EMBED_DIM), lambda j: (0, j, 0)),
        ],
        out_specs=(
            pl.BlockSpec((BATCH, LCHUNK, EMBED_DIM), lambda j: (0, j, 0)),
            pl.BlockSpec((POOL_SIZE, BATCH), lambda j: (0, 0)),
            pl.BlockSpec((BATCH, TOP_K, EMBED_DIM), lambda j: (0, 0, 0)),
            pl.BlockSpec((TOP_K, BATCH), lambda j: (0, 0)),
            pl.BlockSpec((POOL_SIZE,), lambda j: (0,)),
        ),
        out_shape=(
            jax.ShapeDtypeStruct((BATCH, LENGTH, EMBED_DIM), jnp.float32),
            jax.ShapeDtypeStruct((POOL_SIZE, BATCH), jnp.float32),
            jax.ShapeDtypeStruct((BATCH, TOP_K, EMBED_DIM), jnp.float32),
            jax.ShapeDtypeStruct((TOP_K, BATCH), jnp.int32),
            jax.ShapeDtypeStruct((POOL_SIZE,), jnp.float32),
        ),
        scratch_shapes=[
            pltpu.VMEM((BATCH, POOL_SIZE), jnp.float32),
            pltpu.VMEM((POOL_SIZE, EMBED_DIM), jnp.float32),
            pltpu.VMEM((BATCH, TOP_K), jnp.int32),
        ],
    )(cls_features, prompt_key, prompt)
    return (bp, sim.T, keys, idx.T, pool)
